# baseline (speedup 1.0000x reference)
import jax
import jax.numpy as jnp
from jax import lax
from jax.experimental import pallas as pl
from jax.experimental.pallas import tpu as pltpu

_K = 8


def _chunk_counts(xb):
    R, C = xb.shape
    m = jnp.max(xb, axis=1, keepdims=True)
    iota = lax.broadcasted_iota(jnp.int32, (R, C), 1).astype(jnp.float32)
    cand = jnp.where(xb == m, iota, jnp.float32(C))
    vote = jnp.min(cand, axis=1, keepdims=True)
    fo = (iota == vote).astype(jnp.bfloat16)
    ones = jnp.ones((1, R), jnp.bfloat16)
    return lax.dot_general(
        ones, fo, (((1,), (0,)), ((), ())),
        preferred_element_type=jnp.float32,
    )


def _vote_body(x_hbm, out_ref, bufs, acc_ref, sems):
    s = pl.program_id(0)
    nb = pl.num_programs(0)
    K, R, C = bufs.shape
    slot = lax.rem(s, K)

    @pl.when(s == 0)
    def _prologue():
        for k in range(K):
            pltpu.make_async_copy(
                x_hbm.at[pl.ds(k * R, R), :], bufs.at[k], sems.at[k]
            ).start()

    pltpu.make_async_copy(
        x_hbm.at[pl.ds(s * R, R), :], bufs.at[slot], sems.at[slot]
    ).wait()
    cnt = _chunk_counts(bufs[slot])

    @pl.when(s == 0)
    def _init():
        acc_ref[...] = cnt

    @pl.when(s > 0)
    def _acc():
        acc_ref[...] += cnt

    nxt = s + K

    @pl.when(nxt < nb)
    def _issue_next():
        pltpu.make_async_copy(
            x_hbm.at[pl.ds(nxt * R, R), :], bufs.at[slot], sems.at[slot]
        ).start()

    @pl.when(s == nb - 1)
    def _final():
        counts = acc_ref[0, :]
        cm = jnp.max(counts)
        iota1 = lax.iota(jnp.int32, C).astype(jnp.float32)
        cand2 = jnp.where(counts == cm, iota1, jnp.float32(C))
        w = jnp.min(cand2)
        out_ref[0, :] = (iota1 == w).astype(jnp.int32)


def kernel(x):
    N, C = x.shape
    R = 1000 if N % 1000 == 0 else N
    grid = N // R
    ring = min(_K, grid)
    out = pl.pallas_call(
        _vote_body,
        grid=(grid,),
        in_specs=[pl.BlockSpec(memory_space=pltpu.HBM)],
        out_specs=pl.BlockSpec((1, C), lambda i: (0, 0)),
        out_shape=jax.ShapeDtypeStruct((1, C), jnp.int32),
        scratch_shapes=[
            pltpu.VMEM((ring, R, C), jnp.float32),
            pltpu.VMEM((1, C), jnp.float32),
            pltpu.SemaphoreType.DMA((ring,)),
        ],
    )(x)
    return out[0]


# speedup vs baseline: 1.0074x; 1.0074x over previous
import jax
import jax.numpy as jnp
from jax import lax
from jax.experimental import pallas as pl
from jax.experimental.pallas import tpu as pltpu

_K = 4
_S = 5


def _chunk_counts(xb):
    R, C = xb.shape
    m = jnp.max(xb, axis=1, keepdims=True)
    iota = lax.broadcasted_iota(jnp.int32, (R, C), 1).astype(jnp.float32)
    cand = jnp.where(xb == m, iota, jnp.float32(C))
    vote = jnp.min(cand, axis=1, keepdims=True)
    fo = (iota == vote).astype(jnp.bfloat16)
    ones = jnp.ones((1, R), jnp.bfloat16)
    return lax.dot_general(
        ones, fo, (((1,), (0,)), ((), ())),
        preferred_element_type=jnp.float32,
    )


def _make_body(S):
    def _vote_body(x_hbm, out_ref, bufs, acc_ref, sems):
        s = pl.program_id(0)
        nb = pl.num_programs(0)
        K, R, C = bufs.shape
        P = R // S
        slot = lax.rem(s, K)

        def issue(chunk, slot_idx):
            for j in range(S):
                pltpu.make_async_copy(
                    x_hbm.at[pl.ds(chunk * R + j * P, P), :],
                    bufs.at[slot_idx, pl.ds(j * P, P)],
                    sems.at[slot_idx, j],
                ).start()

        @pl.when(s == 0)
        def _prologue():
            for k in range(K):
                issue(k, k)

        for j in range(S):
            pltpu.make_async_copy(
                x_hbm.at[pl.ds(s * R + j * P, P), :],
                bufs.at[slot, pl.ds(j * P, P)],
                sems.at[slot, j],
            ).wait()
        cnt = _chunk_counts(bufs[slot])

        @pl.when(s == 0)
        def _init():
            acc_ref[...] = cnt

        @pl.when(s > 0)
        def _acc():
            acc_ref[...] += cnt

        nxt = s + K

        @pl.when(nxt < nb)
        def _issue_next():
            issue(nxt, slot)

        @pl.when(s == nb - 1)
        def _final():
            counts = acc_ref[0, :]
            cm = jnp.max(counts)
            iota1 = lax.iota(jnp.int32, C).astype(jnp.float32)
            cand2 = jnp.where(counts == cm, iota1, jnp.float32(C))
            w = jnp.min(cand2)
            out_ref[0, :] = (iota1 == w).astype(jnp.int32)

    return _vote_body


def kernel(x):
    N, C = x.shape
    R = 1000 if N % 1000 == 0 else N
    grid = N // R
    ring = min(_K, grid)
    S = _S if (R % _S == 0 and (R // _S) % 8 == 0) else 1
    out = pl.pallas_call(
        _make_body(S),
        grid=(grid,),
        in_specs=[pl.BlockSpec(memory_space=pltpu.HBM)],
        out_specs=pl.BlockSpec((1, C), lambda i: (0, 0)),
        out_shape=jax.ShapeDtypeStruct((1, C), jnp.int32),
        scratch_shapes=[
            pltpu.VMEM((ring, R, C), jnp.float32),
            pltpu.VMEM((1, C), jnp.float32),
            pltpu.SemaphoreType.DMA((ring, S)),
        ],
    )(x)
    return out[0]
